# Initial kernel scaffold; baseline (speedup 1.0000x reference)
import jax
import jax.numpy as jnp
from jax.experimental import pallas as pl


def kernel(x, fc_weight, bias):
    raise NotImplementedError("write your pallas kernel here")


# speedup vs baseline: 1.1487x; 1.1487x over previous
import functools

import jax
import jax.numpy as jnp
from jax import lax
from jax.experimental import pallas as pl
from jax.experimental.pallas import tpu as pltpu
from jax.experimental.pallas import tpu_sc as plsc

_BATCH = 16384
_NUM_FIELDS = 26
_FIELD_SIZE = 100000
_TOTAL_ROWS = _NUM_FIELDS * _FIELD_SIZE
_NC, _NS, _L = 2, 16, 16
_NW = _NC * _NS
_BPW = _BATCH // _NW
_CHUNKS = _BPW // _L
_RPF = _BPW // 128
_ROWS = _NUM_FIELDS * _RPF


def _body(x_hbm, w_hbm, b_hbm, out_hbm, x_v, idx_v, gat_v, out_v, bias_v, sem):
    wid = lax.axis_index("s") * _NC + lax.axis_index("c")
    base = wid * _BPW

    pltpu.sync_copy(x_hbm.at[pl.ds(base * _NUM_FIELDS, _BPW * _NUM_FIELDS)], x_v)
    bias_v[...] = jnp.zeros((_L,), jnp.float32)
    pltpu.sync_copy(b_hbm, bias_v.at[pl.ds(0, 1)])

    lanes = jnp.arange(_L, dtype=jnp.int32)

    def build_field(f, carry):
        off = f * _FIELD_SIZE
        for c in range(_CHUNKS):
            flat = (c * _L + lanes) * _NUM_FIELDS + f
            vals = plsc.load_gather(x_v, [flat])
            idx_v[pl.ds(f * _BPW + c * _L, _L)] = vals + off
        return carry

    lax.fori_loop(0, _NUM_FIELDS, build_field, 0)

    pltpu.async_copy(w_hbm.at[idx_v], gat_v, sem).wait()

    bias_s = jnp.sum(bias_v[...])

    def reduce_chunk(c, carry):
        cb = c * _L
        acc = jnp.zeros((_L,), jnp.float32)
        for f in range(_NUM_FIELDS):
            acc = acc + gat_v[pl.ds(f * _BPW + cb, _L)]
        out_v[pl.ds(cb, _L)] = acc + bias_s
        return carry

    lax.fori_loop(0, _CHUNKS, reduce_chunk, 0)

    pltpu.sync_copy(out_v, out_hbm.at[pl.ds(base, _BPW)])


@functools.cache
def _build():
    mesh = plsc.VectorSubcoreMesh(core_axis_name="c", subcore_axis_name="s")
    return pl.kernel(
        _body,
        out_type=jax.ShapeDtypeStruct((_BATCH,), jnp.float32),
        mesh=mesh,
        scratch_types=[
            pltpu.VMEM((_BPW * _NUM_FIELDS,), jnp.int32),
            pltpu.VMEM((_NUM_FIELDS * _BPW,), jnp.int32),
            pltpu.VMEM((_NUM_FIELDS * _BPW,), jnp.float32),
            pltpu.VMEM((_BPW,), jnp.float32),
            pltpu.VMEM((_L,), jnp.float32),
            pltpu.SemaphoreType.DMA,
        ],
        compiler_params=pltpu.CompilerParams(needs_layout_passes=False),
    )


def kernel(x, fc_weight, bias):
    w = fc_weight.reshape(_TOTAL_ROWS)
    out = _build()(x.reshape(_BATCH * _NUM_FIELDS), w, bias)
    return out.reshape(_BATCH, 1)


# speedup vs baseline: 3.4602x; 3.0124x over previous
import functools

import jax
import jax.numpy as jnp
from jax import lax
from jax.experimental import pallas as pl
from jax.experimental.pallas import tpu as pltpu
from jax.experimental.pallas import tpu_sc as plsc

_BATCH = 16384
_NUM_FIELDS = 26
_FIELD_SIZE = 100000
_TOTAL_ROWS = _NUM_FIELDS * _FIELD_SIZE
_NC, _NS, _L = 2, 16, 16
_NW = _NC * _NS
_BPW = _BATCH // _NW
_CHUNKS = _BPW // _L
_N = _NUM_FIELDS * _BPW

_XROWS = 32
_WPAD = ((_TOTAL_ROWS + 1023) // 1024) * 1024


def _body(x_hbm, w_hbm, b_hbm, out_hbm, idx_v, gat_v, out_v, bias_v, sem):
    wid = lax.axis_index("s") * _NC + lax.axis_index("c")
    base = wid * _BPW

    bias_v[...] = jnp.zeros((_L,), jnp.float32)
    pltpu.sync_copy(b_hbm, bias_v.at[pl.ds(0, 1)])

    for f in range(_NUM_FIELDS):
        pltpu.async_copy(
            x_hbm.at[pl.ds(f * _BATCH + base, _BPW)],
            idx_v.at[pl.ds(f * _BPW, _BPW)],
            sem,
        )
    for f in range(_NUM_FIELDS):
        pltpu.make_async_copy(
            x_hbm.at[pl.ds(f * _BATCH + base, _BPW)],
            idx_v.at[pl.ds(f * _BPW, _BPW)],
            sem,
        ).wait()

    def build_field(f, carry):
        off = f * _FIELD_SIZE
        for c in range(_CHUNKS):
            s = pl.ds(f * _BPW + c * _L, _L)
            idx_v[s] = idx_v[s] + off
        return carry

    lax.fori_loop(1, _NUM_FIELDS, build_field, 0)

    pltpu.async_copy(w_hbm.at[idx_v], gat_v, sem).wait()

    bias_s = jnp.sum(bias_v[...])

    def reduce_chunk(c, carry):
        cb = c * _L
        acc = jnp.zeros((_L,), jnp.float32)
        for f in range(_NUM_FIELDS):
            acc = acc + gat_v[pl.ds(f * _BPW + cb, _L)]
        out_v[pl.ds(cb, _L)] = acc + bias_s
        return carry

    lax.fori_loop(0, _CHUNKS, reduce_chunk, 0)

    pltpu.sync_copy(out_v, out_hbm.at[pl.ds(base, _BPW)])


@functools.cache
def _build():
    mesh = plsc.VectorSubcoreMesh(core_axis_name="c", subcore_axis_name="s")
    return pl.kernel(
        _body,
        out_type=jax.ShapeDtypeStruct((_BATCH,), jnp.float32),
        mesh=mesh,
        scratch_types=[
            pltpu.VMEM((_N,), jnp.int32),
            pltpu.VMEM((_N,), jnp.float32),
            pltpu.VMEM((_BPW,), jnp.float32),
            pltpu.VMEM((_L,), jnp.float32),
            pltpu.SemaphoreType.DMA,
        ],
        compiler_params=pltpu.CompilerParams(needs_layout_passes=False),
    )


def kernel(x, fc_weight, bias):
    xt = jnp.pad(x.T, ((0, _XROWS - _NUM_FIELDS), (0, 0)))
    x_flat = xt.reshape(_XROWS * _BATCH)
    wp = jnp.pad(fc_weight, ((0, _WPAD - _TOTAL_ROWS), (0, 0)))
    w_flat = wp.reshape(_WPAD)
    out = _build()(x_flat, w_flat, bias)
    return out.reshape(_BATCH, 1)


# speedup vs baseline: 3.4687x; 1.0024x over previous
import functools

import jax
import jax.numpy as jnp
from jax import lax
from jax.experimental import pallas as pl
from jax.experimental.pallas import tpu as pltpu
from jax.experimental.pallas import tpu_sc as plsc

_BATCH = 16384
_NUM_FIELDS = 26
_FIELD_SIZE = 100000
_TOTAL_ROWS = _NUM_FIELDS * _FIELD_SIZE
_NC, _NS, _L = 2, 16, 16
_NW = _NC * _NS
_BPW = _BATCH // _NW
_CHUNKS = _BPW // _L
_N = _NUM_FIELDS * _BPW

_XROWS = 32
_WPAD = ((_TOTAL_ROWS + 1023) // 1024) * 1024


def _body(x_hbm, w_hbm, b_hbm, out_hbm, idx_v, gat_v, out_v, bias_v, sem):
    wid = lax.axis_index("s") * _NC + lax.axis_index("c")
    base = wid * _BPW

    bias_v[...] = jnp.zeros((_L,), jnp.float32)
    pltpu.sync_copy(b_hbm, bias_v.at[pl.ds(0, 1)])

    for f in range(_NUM_FIELDS):
        pltpu.async_copy(
            x_hbm.at[pl.ds(f * _BATCH + base, _BPW)],
            idx_v.at[pl.ds(f * _BPW, _BPW)],
            sem,
        )
    for f in range(_NUM_FIELDS):
        pltpu.make_async_copy(
            x_hbm.at[pl.ds(f * _BATCH + base, _BPW)],
            idx_v.at[pl.ds(f * _BPW, _BPW)],
            sem,
        ).wait()

    pltpu.async_copy(w_hbm.at[idx_v], gat_v, sem).wait()

    bias_s = jnp.sum(bias_v[...])

    def reduce_chunk(c, carry):
        cb = c * _L
        acc = jnp.zeros((_L,), jnp.float32)
        for f in range(_NUM_FIELDS):
            acc = acc + gat_v[pl.ds(f * _BPW + cb, _L)]
        out_v[pl.ds(cb, _L)] = acc + bias_s
        return carry

    lax.fori_loop(0, _CHUNKS, reduce_chunk, 0)

    pltpu.sync_copy(out_v, out_hbm.at[pl.ds(base, _BPW)])


@functools.cache
def _build():
    mesh = plsc.VectorSubcoreMesh(core_axis_name="c", subcore_axis_name="s")
    return pl.kernel(
        _body,
        out_type=jax.ShapeDtypeStruct((_BATCH,), jnp.float32),
        mesh=mesh,
        scratch_types=[
            pltpu.VMEM((_N,), jnp.int32),
            pltpu.VMEM((_N,), jnp.float32),
            pltpu.VMEM((_BPW,), jnp.float32),
            pltpu.VMEM((_L,), jnp.float32),
            pltpu.SemaphoreType.DMA,
        ],
        compiler_params=pltpu.CompilerParams(needs_layout_passes=False),
    )


def kernel(x, fc_weight, bias):
    offs = jnp.arange(_NUM_FIELDS, dtype=jnp.int32) * _FIELD_SIZE
    xt = jnp.pad((x + offs[None, :]).T, ((0, _XROWS - _NUM_FIELDS), (0, 0)))
    x_flat = xt.reshape(_XROWS * _BATCH)
    wp = jnp.pad(fc_weight, ((0, _WPAD - _TOTAL_ROWS), (0, 0)))
    w_flat = wp.reshape(_WPAD)
    out = _build()(x_flat, w_flat, bias)
    return out.reshape(_BATCH, 1)
